# baseline (speedup 1.0000x reference)
import jax
import jax.numpy as jnp
import numpy as np
from jax.experimental import pallas as pl
from jax.experimental.pallas import tpu as pltpu

_B, _N, _V = 64, 1024, 1024
_BN = _B * _N
_ROWS = 2048
_NBLK = _BN // _ROWS
_RT = _ROWS // 128
_SUBL = _BN // 128

_TEMP = 4.5 * (1.0 - 0.5)
_GAMMA = float(np.cos(0.5 * np.pi / 2.0))


def _tree(vals, op):
    while len(vals) > 1:
        vals = [op(vals[i], vals[i + 1]) for i in range(0, len(vals), 2)]
    return vals[0]


def _select(conf, mk_full):
    m_total = jnp.sum(mk_full)
    n_mask = jnp.ceil(
        jnp.float32(_GAMMA) * m_total.astype(jnp.float32)).astype(jnp.int32)

    b = jax.lax.bitcast_convert_type(conf, jnp.int32)
    key = jnp.where(b >= 0, b, (~b) + jnp.int32(-2147483648))

    def body(_, carry):
        lo, hi = carry
        mid = (lo >> 1) + (hi >> 1) + (lo & hi & 1)
        cnt = jnp.sum((key <= mid).astype(jnp.int32))
        go_left = cnt >= n_mask
        return (jnp.where(go_left, lo, mid + 1), jnp.where(go_left, mid, hi))

    t, _ = jax.lax.fori_loop(
        0, 32, body, (jnp.int32(-(2**31)), jnp.int32(2**31 - 1)))

    cnt_less = jnp.sum((key < t).astype(jnp.int32))
    eq = (key == t).astype(jnp.float32)
    jj = jax.lax.broadcasted_iota(jnp.int32, (128, 128), 1)
    kk = jax.lax.broadcasted_iota(jnp.int32, (128, 128), 0)
    u_tri = (kk < jj).astype(jnp.float32)
    within = jnp.dot(eq, u_tri, preferred_element_type=jnp.float32)
    rows = jnp.sum(eq, axis=1, keepdims=True)
    rr = jax.lax.broadcasted_iota(jnp.int32, (_SUBL, _SUBL), 0)
    cc = jax.lax.broadcasted_iota(jnp.int32, (_SUBL, _SUBL), 1)
    l_tri = (cc < rr).astype(jnp.float32)
    rowpre = jnp.dot(l_tri, rows, preferred_element_type=jnp.float32)
    prefix = (rowpre + within).astype(jnp.int32)
    sel = (key < t) | ((key == t) & ((cnt_less + prefix) < n_mask))
    return sel.astype(jnp.int32)


def _fused_kernel(x_ref, z_ref, mk_ref, g_ref, mkfull_ref,
                  zp_ref, conf_ref, sel_ref, cacc_ref):
    i = pl.program_id(0)
    x = x_ref[...]
    m = jnp.max(x, axis=1, keepdims=True)
    lane = jax.lax.broadcasted_iota(jnp.int32, (_ROWS, 128), 1)
    big = jnp.int32(_V)
    es, idxs = [], []
    for c in range(_V // 128):
        xc = x[:, c * 128:(c + 1) * 128]
        es.append(jnp.exp(xc - m))
        idxs.append(jnp.where(xc == m, lane + c * 128, big))
    s_part = _tree(es, jnp.add)
    i_part = _tree(idxs, jnp.minimum)
    s_row = jnp.sum(s_part.T, axis=0)
    am_row = jnp.min(i_part.T, axis=0)
    maxp = 1.0 / s_row
    mk = mk_ref[0, 0, :] != 0
    zp_ref[0, 0, :] = jnp.where(mk, am_row, z_ref[0, 0, :])
    conf = jnp.where(mk, maxp + jnp.float32(_TEMP) * g_ref[0, 0, :], jnp.inf)
    conf_ref[0, 0, :] = conf
    cacc_ref[pl.ds(i * _RT, _RT), :] = conf.reshape(_RT, 128)

    @pl.when(i == _NBLK - 1)
    def _run_select():
        sel_ref[...] = _select(cacc_ref[...], mkfull_ref[...])


def kernel(logits, z_indices, mask_b):
    B, N, V = logits.shape
    x = logits.reshape(B * N, V)
    mk_flat = mask_b.reshape(-1).astype(jnp.int32)
    z_flat = z_indices.reshape(-1)
    e = jax.random.exponential(jax.random.key(42), (B, N), dtype=jnp.float32)
    g_flat = (-jnp.log(e)).reshape(-1)

    small = lambda a: a.reshape(_NBLK, 1, _ROWS)
    small_spec = pl.BlockSpec((1, 1, _ROWS), lambda i: (i, 0, 0))
    full_spec = pl.BlockSpec((_SUBL, 128), lambda i: (0, 0))
    zp, conf, sel = pl.pallas_call(
        _fused_kernel,
        grid=(_NBLK,),
        in_specs=[
            pl.BlockSpec((_ROWS, _V), lambda i: (i, 0)),
            small_spec, small_spec, small_spec, full_spec,
        ],
        out_specs=[small_spec, small_spec, full_spec],
        out_shape=[
            jax.ShapeDtypeStruct((_NBLK, 1, _ROWS), jnp.int32),
            jax.ShapeDtypeStruct((_NBLK, 1, _ROWS), jnp.float32),
            jax.ShapeDtypeStruct((_SUBL, 128), jnp.int32),
        ],
        scratch_shapes=[pltpu.VMEM((_SUBL, 128), jnp.float32)],
        compiler_params=pltpu.CompilerParams(
            dimension_semantics=("arbitrary",)),
    )(x, small(z_flat), small(mk_flat), small(g_flat),
      mk_flat.reshape(_SUBL, 128))

    z_indices_predict = zp.reshape(B, N)
    new_mask_b = sel.reshape(B, N).astype(bool)
    return (z_indices_predict, new_mask_b, conf.reshape(_BN))


# speedup vs baseline: 1.0551x; 1.0551x over previous
import jax
import jax.numpy as jnp
import numpy as np
from jax.experimental import pallas as pl
from jax.experimental.pallas import tpu as pltpu

_B, _N, _V = 64, 1024, 1024
_BN = _B * _N
_ROWS = 2048
_NBLK = _BN // _ROWS
_RT = _ROWS // 128
_SUBL = _BN // 128

_TEMP = 4.5 * (1.0 - 0.5)
_GAMMA = float(np.cos(0.5 * np.pi / 2.0))


def _tree(vals, op):
    while len(vals) > 1:
        vals = [op(vals[i], vals[i + 1]) for i in range(0, len(vals), 2)]
    return vals[0]


def _gumbel_row(base):
    lo = jax.lax.broadcasted_iota(jnp.uint32, (1, _ROWS), 1) + base
    ks = [jnp.uint32(0), jnp.uint32(42), jnp.uint32(0x1BD11BDA ^ 42)]
    x0 = jnp.full((1, _ROWS), ks[0], jnp.uint32)
    x1 = lo + ks[1]
    rot = (13, 15, 26, 6, 17, 29, 16, 24)
    for grp in range(5):
        for r in rot[0:4] if grp % 2 == 0 else rot[4:8]:
            x0 = x0 + x1
            x1 = (x1 << r) | (x1 >> (32 - r))
            x1 = x1 ^ x0
        x0 = x0 + ks[(grp + 1) % 3]
        x1 = x1 + ks[(grp + 2) % 3] + jnp.uint32(grp + 1)
    bits = x0 ^ x1
    fl = jax.lax.bitcast_convert_type(
        (bits >> 9) | jnp.uint32(0x3F800000), jnp.float32)
    e = -jnp.log1p(-(fl - 1.0))
    return -jnp.log(e).reshape(_ROWS)


def _select(conf, mk_full):
    m_total = jnp.sum(mk_full)
    n_mask = jnp.ceil(
        jnp.float32(_GAMMA) * m_total.astype(jnp.float32)).astype(jnp.int32)

    b = jax.lax.bitcast_convert_type(conf, jnp.int32)
    key = jnp.where(b >= 0, b, (~b) + jnp.int32(-2147483648))

    def body(_, carry):
        lo, hi = carry
        mid = (lo >> 1) + (hi >> 1) + (lo & hi & 1)
        cnt = jnp.sum((key <= mid).astype(jnp.int32))
        go_left = cnt >= n_mask
        return (jnp.where(go_left, lo, mid + 1), jnp.where(go_left, mid, hi))

    t, _ = jax.lax.fori_loop(
        0, 32, body, (jnp.int32(-(2**31)), jnp.int32(2**31 - 1)))

    cnt_less = jnp.sum((key < t).astype(jnp.int32))
    eq = (key == t).astype(jnp.float32)
    jj = jax.lax.broadcasted_iota(jnp.int32, (128, 128), 1)
    kk = jax.lax.broadcasted_iota(jnp.int32, (128, 128), 0)
    u_tri = (kk < jj).astype(jnp.float32)
    within = jnp.dot(eq, u_tri, preferred_element_type=jnp.float32)
    rows = jnp.sum(eq, axis=1, keepdims=True)
    rr = jax.lax.broadcasted_iota(jnp.int32, (_SUBL, _SUBL), 0)
    cc = jax.lax.broadcasted_iota(jnp.int32, (_SUBL, _SUBL), 1)
    l_tri = (cc < rr).astype(jnp.float32)
    rowpre = jnp.dot(l_tri, rows, preferred_element_type=jnp.float32)
    prefix = (rowpre + within).astype(jnp.int32)
    sel = (key < t) | ((key == t) & ((cnt_less + prefix) < n_mask))
    return sel.astype(jnp.int32)


def _fused_kernel(x_ref, z_ref, mk_ref, mkfull_ref,
                  zp_ref, conf_ref, sel_ref, cacc_ref):
    i = pl.program_id(0)
    x = x_ref[...]
    m = jnp.max(x, axis=1, keepdims=True)
    lane = jax.lax.broadcasted_iota(jnp.int32, (_ROWS, 128), 1)
    big = jnp.int32(_V)
    es, idxs = [], []
    for c in range(_V // 128):
        xc = x[:, c * 128:(c + 1) * 128]
        es.append(jnp.exp(xc - m))
        idxs.append(jnp.where(xc == m, lane + c * 128, big))
    s_part = _tree(es, jnp.add)
    i_part = _tree(idxs, jnp.minimum)
    s_row = jnp.sum(s_part.T, axis=0)
    am_row = jnp.min(i_part.T, axis=0)
    maxp = 1.0 / s_row
    mk = mk_ref[0, 0, :] != 0
    zp_ref[0, 0, :] = jnp.where(mk, am_row, z_ref[0, 0, :])
    g_row = _gumbel_row((i * _ROWS).astype(jnp.uint32))
    conf = jnp.where(mk, maxp + jnp.float32(_TEMP) * g_row, jnp.inf)
    conf_ref[0, 0, :] = conf
    cacc_ref[pl.ds(i * _RT, _RT), :] = conf.reshape(_RT, 128)

    @pl.when(i == _NBLK - 1)
    def _run_select():
        sel_ref[...] = _select(cacc_ref[...], mkfull_ref[...])


def kernel(logits, z_indices, mask_b):
    B, N, V = logits.shape
    x = logits.reshape(B * N, V)
    mk_flat = mask_b.reshape(-1).astype(jnp.int32)
    z_flat = z_indices.reshape(-1)

    small = lambda a: a.reshape(_NBLK, 1, _ROWS)
    small_spec = pl.BlockSpec((1, 1, _ROWS), lambda i: (i, 0, 0))
    full_spec = pl.BlockSpec((_SUBL, 128), lambda i: (0, 0))
    zp, conf, sel = pl.pallas_call(
        _fused_kernel,
        grid=(_NBLK,),
        in_specs=[
            pl.BlockSpec((_ROWS, _V), lambda i: (i, 0)),
            small_spec, small_spec, full_spec,
        ],
        out_specs=[small_spec, small_spec, full_spec],
        out_shape=[
            jax.ShapeDtypeStruct((_NBLK, 1, _ROWS), jnp.int32),
            jax.ShapeDtypeStruct((_NBLK, 1, _ROWS), jnp.float32),
            jax.ShapeDtypeStruct((_SUBL, 128), jnp.int32),
        ],
        scratch_shapes=[pltpu.VMEM((_SUBL, 128), jnp.float32)],
        compiler_params=pltpu.CompilerParams(
            dimension_semantics=("arbitrary",)),
    )(x, small(z_flat), small(mk_flat), mk_flat.reshape(_SUBL, 128))

    z_indices_predict = zp.reshape(B, N)
    new_mask_b = sel.reshape(B, N).astype(bool)
    return (z_indices_predict, new_mask_b, conf.reshape(_BN))


# speedup vs baseline: 1.0564x; 1.0012x over previous
import jax
import jax.numpy as jnp
import numpy as np
from jax.experimental import pallas as pl
from jax.experimental.pallas import tpu as pltpu

_B, _N, _V = 64, 1024, 1024
_BN = _B * _N
_ROWS = 2048
_NBLK = _BN // _ROWS
_RT = _ROWS // 128
_SUBL = _BN // 128

_TEMP = 4.5 * (1.0 - 0.5)
_GAMMA = float(np.cos(0.5 * np.pi / 2.0))


def _tree(vals, op):
    while len(vals) > 1:
        vals = [op(vals[i], vals[i + 1]) for i in range(0, len(vals), 2)]
    return vals[0]


def _gumbel_row(base):
    lo = jax.lax.broadcasted_iota(jnp.uint32, (1, _ROWS), 1) + base
    ks = [jnp.uint32(0), jnp.uint32(42), jnp.uint32(0x1BD11BDA ^ 42)]
    x0 = jnp.full((1, _ROWS), ks[0], jnp.uint32)
    x1 = lo + ks[1]
    rot = (13, 15, 26, 6, 17, 29, 16, 24)
    for grp in range(5):
        for r in rot[0:4] if grp % 2 == 0 else rot[4:8]:
            x0 = x0 + x1
            x1 = (x1 << r) | (x1 >> (32 - r))
            x1 = x1 ^ x0
        x0 = x0 + ks[(grp + 1) % 3]
        x1 = x1 + ks[(grp + 2) % 3] + jnp.uint32(grp + 1)
    bits = x0 ^ x1
    fl = jax.lax.bitcast_convert_type(
        (bits >> 9) | jnp.uint32(0x3F800000), jnp.float32)
    e = -jnp.log1p(-(fl - 1.0))
    return -jnp.log(e).reshape(_ROWS)


def _select(conf, mk_full):
    m_total = jnp.sum(mk_full.astype(jnp.int32))
    n_mask = jnp.ceil(
        jnp.float32(_GAMMA) * m_total.astype(jnp.float32)).astype(jnp.int32)

    b = jax.lax.bitcast_convert_type(conf, jnp.int32)
    key = jnp.where(b >= 0, b, (~b) + jnp.int32(-2147483648))

    def body(_, carry):
        lo, hi = carry
        mid = (lo >> 1) + (hi >> 1) + (lo & hi & 1)
        cnt = jnp.sum((key <= mid).astype(jnp.int32))
        go_left = cnt >= n_mask
        return (jnp.where(go_left, lo, mid + 1), jnp.where(go_left, mid, hi))

    t, _ = jax.lax.fori_loop(
        0, 32, body, (jnp.int32(-(2**31)), jnp.int32(2**31 - 1)))

    cnt_less = jnp.sum((key < t).astype(jnp.int32))
    eq = (key == t).astype(jnp.float32)
    jj = jax.lax.broadcasted_iota(jnp.int32, (128, 128), 1)
    kk = jax.lax.broadcasted_iota(jnp.int32, (128, 128), 0)
    u_tri = (kk < jj).astype(jnp.float32)
    within = jnp.dot(eq, u_tri, preferred_element_type=jnp.float32)
    rows = jnp.sum(eq, axis=1, keepdims=True)
    rr = jax.lax.broadcasted_iota(jnp.int32, (_SUBL, _SUBL), 0)
    cc = jax.lax.broadcasted_iota(jnp.int32, (_SUBL, _SUBL), 1)
    l_tri = (cc < rr).astype(jnp.float32)
    rowpre = jnp.dot(l_tri, rows, preferred_element_type=jnp.float32)
    prefix = (rowpre + within).astype(jnp.int32)
    return (key < t) | ((key == t) & ((cnt_less + prefix) < n_mask))


def _fused_kernel(x_ref, z_ref, mk_ref, mkfull_ref,
                  zp_ref, conf_ref, sel_ref, cacc_ref):
    i = pl.program_id(0)
    x = x_ref[...]
    m = jnp.max(x, axis=1, keepdims=True)
    lane = jax.lax.broadcasted_iota(jnp.int32, (_ROWS, 128), 1)
    big = jnp.int32(_V)
    es, idxs = [], []
    for c in range(_V // 128):
        xc = x[:, c * 128:(c + 1) * 128]
        es.append(jnp.exp(xc - m))
        idxs.append(jnp.where(xc == m, lane + c * 128, big))
    s_part = _tree(es, jnp.add)
    i_part = _tree(idxs, jnp.minimum)
    s_row = jnp.sum(s_part.T, axis=0)
    am_row = jnp.min(i_part.T, axis=0)
    maxp = 1.0 / s_row
    mk = mk_ref[0, 0, :]
    zp_ref[0, 0, :] = jnp.where(mk, am_row, z_ref[0, 0, :])
    g_row = _gumbel_row((i * _ROWS).astype(jnp.uint32))
    conf = jnp.where(mk, maxp + jnp.float32(_TEMP) * g_row, jnp.inf)
    conf_ref[0, 0, :] = conf
    cacc_ref[pl.ds(i * _RT, _RT), :] = conf.reshape(_RT, 128)

    @pl.when(i == _NBLK - 1)
    def _run_select():
        sel_ref[...] = _select(cacc_ref[...], mkfull_ref[...])


def kernel(logits, z_indices, mask_b):
    B, N, V = logits.shape
    x = logits.reshape(B * N, V)
    mk_flat = mask_b.reshape(-1)
    z_flat = z_indices.reshape(-1)

    small = lambda a: a.reshape(_NBLK, 1, _ROWS)
    small_spec = pl.BlockSpec((1, 1, _ROWS), lambda i: (i, 0, 0))
    full_spec = pl.BlockSpec((_SUBL, 128), lambda i: (0, 0))
    zp, conf, sel = pl.pallas_call(
        _fused_kernel,
        grid=(_NBLK,),
        in_specs=[
            pl.BlockSpec((_ROWS, _V), lambda i: (i, 0)),
            small_spec, small_spec, full_spec,
        ],
        out_specs=[small_spec, small_spec, full_spec],
        out_shape=[
            jax.ShapeDtypeStruct((_NBLK, 1, _ROWS), jnp.int32),
            jax.ShapeDtypeStruct((_NBLK, 1, _ROWS), jnp.float32),
            jax.ShapeDtypeStruct((_SUBL, 128), jnp.bool_),
        ],
        scratch_shapes=[pltpu.VMEM((_SUBL, 128), jnp.float32)],
        compiler_params=pltpu.CompilerParams(
            dimension_semantics=("arbitrary",)),
    )(x, small(z_flat), small(mk_flat), mk_flat.reshape(_SUBL, 128))

    z_indices_predict = zp.reshape(B, N)
    new_mask_b = sel.reshape(B, N)
    return (z_indices_predict, new_mask_b, conf.reshape(_BN))
